# baseline (device time: 674479 ns/iter reference)
import jax
import jax.numpy as jnp
from jax import lax
from jax.experimental import pallas as pl
from jax.experimental.pallas import tpu as pltpu

S = 2048
N = 8192
S_OUT = 1024
CB = 1024
N_CHUNK = N // CB

BM, BN, BK = 512, 2048, 1024


def _matmul_body(a_ref, b_ref, c_ref):
    k = pl.program_id(2)

    @pl.when(k == 0)
    def _():
        c_ref[...] = jnp.zeros_like(c_ref)

    c_ref[...] += lax.dot_general(
        a_ref[...], b_ref[...], (((1,), (0,)), ((), ())),
        preferred_element_type=jnp.float32)


def _matmul(a, b):
    m, k = a.shape
    _, n = b.shape
    return pl.pallas_call(
        _matmul_body,
        grid=(m // BM, n // BN, k // BK),
        in_specs=[
            pl.BlockSpec((BM, BK), lambda i, j, kk: (i, kk)),
            pl.BlockSpec((BK, BN), lambda i, j, kk: (kk, j)),
        ],
        out_specs=pl.BlockSpec((BM, BN), lambda i, j, kk: (i, j)),
        out_shape=jax.ShapeDtypeStruct((m, n), jnp.float32),
        compiler_params=pltpu.CompilerParams(
            dimension_semantics=("parallel", "parallel", "arbitrary")),
    )(a, b)


def _comm_body(p_ref, out_ref, recv_buf, send_sems, recv_sems):
    j = pl.program_id(0)
    my_x = lax.axis_index("x")
    my_y = lax.axis_index("y")
    partner = (1 - my_x, my_y)

    @pl.when(j == 0)
    def _():
        barrier = pltpu.get_barrier_semaphore()
        pl.semaphore_signal(barrier, inc=1, device_id=partner,
                            device_id_type=pl.DeviceIdType.MESH)
        pl.semaphore_wait(barrier, 1)

    slot = lax.rem(j, 2)
    rdma = pltpu.make_async_remote_copy(
        src_ref=p_ref.at[pl.ds((1 - my_x) * S_OUT, S_OUT), :],
        dst_ref=recv_buf.at[slot],
        send_sem=send_sems.at[j],
        recv_sem=recv_sems.at[j],
        device_id=partner,
        device_id_type=pl.DeviceIdType.MESH,
    )
    rdma.start()
    rdma.wait()
    out_ref[...] = p_ref[pl.ds(my_x * S_OUT, S_OUT), :] + recv_buf[slot]


def kernel(O, Wo):
    b, s, h, d = O.shape
    a = O.reshape(s, h * d)
    p = _matmul(a, Wo)

    out = pl.pallas_call(
        _comm_body,
        grid=(N_CHUNK,),
        in_specs=[pl.BlockSpec((S, CB), lambda j: (0, j))],
        out_specs=pl.BlockSpec((S_OUT, CB), lambda j: (0, j)),
        out_shape=jax.ShapeDtypeStruct((S_OUT, N), jnp.float32),
        scratch_shapes=[
            pltpu.VMEM((2, S_OUT, CB), jnp.float32),
            pltpu.SemaphoreType.DMA((N_CHUNK,)),
            pltpu.SemaphoreType.DMA((N_CHUNK,)),
        ],
        compiler_params=pltpu.CompilerParams(
            dimension_semantics=("arbitrary",),
            collective_id=0),
    )(p)
    return out.reshape(1, S_OUT, N)


# device time: 445416 ns/iter; 1.5143x vs baseline; 1.5143x over previous
import jax
import jax.numpy as jnp
from jax import lax
from jax.experimental import pallas as pl
from jax.experimental.pallas import tpu as pltpu

S = 2048
N = 8192
K_LOC = 4096
S_OUT = 1024
CB = 1024
NB = N // CB
BK = 512
KB = K_LOC // BK
NSEND = NB // 2

MESH = pl.DeviceIdType.MESH


def _fused_body(a_ref, b_ref, p_ref, po_ref,
                stage, x_send, x_recv, y_send, y_recv):
    n = pl.program_id(0)
    k = pl.program_id(1)
    my_x = lax.axis_index("x")
    my_y = lax.axis_index("y")
    x_partner = (1 - my_x, my_y)
    y_partner = (my_x, 1 - my_y)

    @pl.when((n == 0) & (k == 0))
    def _():
        bar = pltpu.get_barrier_semaphore()
        pl.semaphore_signal(bar, inc=1, device_id=x_partner,
                            device_id_type=MESH)
        pl.semaphore_signal(bar, inc=1, device_id=y_partner,
                            device_id_type=MESH)
        pl.semaphore_wait(bar, 2)

    @pl.when(k == 0)
    def _():
        p_ref[...] = jnp.zeros_like(p_ref)

    p_ref[...] += lax.dot_general(
        a_ref[...], b_ref[...], (((1,), (0,)), ((), ())),
        preferred_element_type=jnp.float32)

    def x_desc(jj, blk):
        return pltpu.make_async_remote_copy(
            src_ref=stage.at[jj],
            dst_ref=po_ref.at[:, pl.ds(blk * CB, CB)],
            send_sem=x_send.at[jj],
            recv_sem=x_recv.at[jj],
            device_id=x_partner,
            device_id_type=MESH)

    def y_out_desc(jj):
        blk = 2 * jj + my_y
        return pltpu.make_async_remote_copy(
            src_ref=po_ref.at[:, pl.ds(blk * CB, CB)],
            dst_ref=po_ref.at[:, pl.ds(blk * CB, CB)],
            send_sem=y_send.at[jj],
            recv_sem=y_recv.at[jj],
            device_id=y_partner,
            device_id_type=MESH)

    def y_in_desc(jj):
        blk = 2 * jj + (1 - my_y)
        return pltpu.make_async_remote_copy(
            src_ref=po_ref.at[:, pl.ds(blk * CB, CB)],
            dst_ref=po_ref.at[:, pl.ds(blk * CB, CB)],
            send_sem=y_send.at[jj],
            recv_sem=y_recv.at[jj],
            device_id=y_partner,
            device_id_type=MESH)

    k_last = KB - 1

    @pl.when((k == k_last) & (n % 2 == my_y))
    def _():
        jj = n // 2
        stage[jj] = p_ref[pl.ds((1 - my_x) * S_OUT, S_OUT), :]
        x_desc(jj, n).start()

    @pl.when((k == k_last) & (n >= 3) & (n % 2 == 1))
    def _():
        jj = (n - 3) // 2
        x_desc(jj, 2 * jj + my_y).wait_recv()
        y_out_desc(jj).start()

    @pl.when((k == k_last) & (n == NB - 1))
    def _():
        jj = NSEND - 1
        x_desc(jj, 2 * jj + my_y).wait_recv()
        y_out_desc(jj).start()
        for t in range(NSEND):
            x_desc(t, 2 * t + my_y).wait_send()
        for t in range(NSEND):
            y_out_desc(t).wait_send()
        for t in range(NSEND):
            y_in_desc(t).wait_recv()


def _add_body(p_hbm, po_ref, out_ref, keep, sem):
    j = pl.program_id(0)
    my_x = lax.axis_index("x")
    cp = pltpu.make_async_copy(
        p_hbm.at[pl.ds(my_x * S_OUT, S_OUT), pl.ds(j * CB, CB)], keep, sem)
    cp.start()
    cp.wait()
    out_ref[...] = keep[...] + po_ref[...]


def kernel(O, Wo):
    b, s, h, d = O.shape
    a2 = O.reshape(s, h * d)

    p, po = pl.pallas_call(
        _fused_body,
        grid=(NB, KB),
        in_specs=[
            pl.BlockSpec((S, BK), lambda n, k: (0, k)),
            pl.BlockSpec((BK, CB), lambda n, k: (k, n)),
        ],
        out_specs=[
            pl.BlockSpec((S, CB), lambda n, k: (0, n)),
            pl.BlockSpec(memory_space=pl.ANY),
        ],
        out_shape=[
            jax.ShapeDtypeStruct((S, N), jnp.float32),
            jax.ShapeDtypeStruct((S_OUT, N), jnp.float32),
        ],
        scratch_shapes=[
            pltpu.VMEM((NSEND, S_OUT, CB), jnp.float32),
            pltpu.SemaphoreType.DMA((NSEND,)),
            pltpu.SemaphoreType.DMA((NSEND,)),
            pltpu.SemaphoreType.DMA((NSEND,)),
            pltpu.SemaphoreType.DMA((NSEND,)),
        ],
        compiler_params=pltpu.CompilerParams(
            dimension_semantics=("arbitrary", "arbitrary"),
            vmem_limit_bytes=56 * 1024 * 1024,
            collective_id=0),
    )(a2, Wo)

    out = pl.pallas_call(
        _add_body,
        grid=(NB,),
        in_specs=[
            pl.BlockSpec(memory_space=pl.ANY),
            pl.BlockSpec((S_OUT, CB), lambda j: (0, j)),
        ],
        out_specs=pl.BlockSpec((S_OUT, CB), lambda j: (0, j)),
        out_shape=jax.ShapeDtypeStruct((S_OUT, N), jnp.float32),
        scratch_shapes=[
            pltpu.VMEM((S_OUT, CB), jnp.float32),
            pltpu.SemaphoreType.DMA,
        ],
    )(p, po)
    return out.reshape(1, S_OUT, N)
